# baseline (speedup 1.0000x reference)
import functools

import jax
import jax.numpy as jnp
from jax import lax
from jax.experimental import pallas as pl
from jax.experimental.pallas import tpu as pltpu
from jax.experimental.pallas import tpu_sc as plsc

N = 10000
E = 160000
D_IN = 256
D_H = 256
D_OUT = 64

NP = 10240
EP = 163840
EROWS = EP // 128
DQ = 64

_mesh = plsc.VectorSubcoreMesh(core_axis_name="c", subcore_axis_name="s")
_sc_params = pltpu.CompilerParams(use_tc_tiling_on_sc=False)


@functools.partial(
    pl.kernel,
    out_type=jax.ShapeDtypeStruct((2, NP), jnp.float32),
    mesh=_mesh,
    compiler_params=_sc_params,
    scratch_types=[
        pltpu.VMEM((4, 128), jnp.int32),
        pltpu.VMEM((128,), jnp.float32),
        pltpu.VMEM((640,), jnp.float32),
        pltpu.VMEM_SHARED((NP,), jnp.float32),
    ],
)
def _deg_kernel(edge_hbm, z1d_hbm, out_hbm, idxv, ones, vbuf, acc):
    c = lax.axis_index("c")
    s = lax.axis_index("s")
    pltpu.sync_copy(z1d_hbm.at[pl.ds(0, 640)], vbuf)
    pltpu.sync_copy(vbuf, acc.at[pl.ds(s * 640, 640)])
    for k in range(8):
        ones[pl.ds(k * 16, 16)] = jnp.ones((16,), jnp.float32)
    plsc.subcore_barrier()

    def body(i, carry):
        pltpu.sync_copy(edge_hbm.at[c, pl.ds(s * 80 + i * 4, 4)], idxv)
        for j in range(4):
            pltpu.sync_copy(ones, acc.at[idxv.at[j]], add=True)
        return carry

    lax.fori_loop(0, 20, body, 0)
    plsc.subcore_barrier()
    pltpu.sync_copy(acc.at[pl.ds(s * 640, 640)], vbuf)
    pltpu.sync_copy(vbuf, out_hbm.at[c, pl.ds(s * 640, 640)])


def _agg_pass(table_hbm, zdrain_hbm, srcb, dstb, rows8, acc, gsem, tsem,
              nslots):
    def fire_gather(slot, b):
        pltpu.async_copy(table_hbm.at[srcb.at[slot]], rows8[b], gsem.at[b])

    def fire_scatter(slot, b):
        pltpu.async_copy(rows8[b], acc.at[dstb.at[slot]], tsem.at[b],
                         add=True)

    def drain(b, sem):
        pltpu.make_async_copy(zdrain_hbm.at[pl.ds(0, 128)], rows8[b],
                              sem.at[b]).wait()

    for b in range(4):
        fire_gather(b, b)

    def body(outer, carry):
        for b in range(8):
            it = outer * 8 + b
            drain(b, gsem)
            fire_scatter(it, b)
            b2 = (b + 4) % 8

            @pl.when(it >= 4)
            def _():
                drain(b2, tsem)

            @pl.when(it + 4 < nslots)
            def _():
                fire_gather(it + 4, b2)
        return carry

    lax.fori_loop(0, nslots // 8, body, 0)
    for b in (4, 5, 6, 7):
        drain(b, tsem)


def _zero_acc(zeros_hbm, zb, acc, s):
    pltpu.sync_copy(zeros_hbm.at[pl.ds(0, 128)], zb)
    for z in range(5):
        pltpu.sync_copy(zb, acc.at[pl.ds(s * 640 + z * 128, 128)])


def _readout(acc, zb, out_hbm, s, out_base):
    for z in range(5):
        pltpu.sync_copy(acc.at[pl.ds(s * 640 + z * 128, 128)], zb)
        pltpu.sync_copy(
            zb, out_hbm.at[pl.ds(out_base + s * 640 + z * 128, 128)])


@functools.partial(
    pl.kernel,
    out_type=jax.ShapeDtypeStruct((4 * NP, 128), jnp.float32),
    mesh=_mesh,
    compiler_params=_sc_params,
    scratch_types=[
        pltpu.VMEM((40, 128), jnp.int32),
        pltpu.VMEM((40, 128), jnp.int32),
        pltpu.VMEM((128, 128), jnp.float32),
        pltpu.VMEM((128, 128), jnp.float32),
        pltpu.VMEM_SHARED((NP, 128), jnp.float32),
        pltpu.SemaphoreType.DMA((2,)),
        pltpu.SemaphoreType.DMA((2,)),
    ],
)
def _agg1_kernel(table_hbm, src4_hbm, dst2_hbm, z128_hbm, out_hbm,
                 srcb, dstb, r0, r1, acc, gsem, tsem):
    c = lax.axis_index("c")
    s = lax.axis_index("s")
    rows2 = [r0, r1]

    def fire_gather(slot, b):
        pltpu.async_copy(table_hbm.at[srcb.at[slot]], rows2[b], gsem.at[b])

    def drain(b, sem):
        pltpu.make_async_copy(table_hbm.at[pl.ds(0, 128)], rows2[b],
                              sem.at[b]).wait()

    for h in range(2):
        pltpu.sync_copy(z128_hbm, r0)
        for z in range(5):
            pltpu.sync_copy(r0, acc.at[pl.ds(s * 640 + z * 128, 128)])
        base = c * 640 + s * 40
        pltpu.sync_copy(src4_hbm.at[h, pl.ds(base, 40)], srcb)
        pltpu.sync_copy(dst2_hbm.at[pl.ds(base, 40)], dstb)
        plsc.subcore_barrier()
        fire_gather(0, 0)

        def body(outer, carry):
            for b in range(2):
                it = outer * 2 + b
                drain(b, gsem)
                pltpu.async_copy(rows2[b], acc.at[dstb.at[it]],
                                 tsem.at[b], add=True)
                b2 = 1 - b

                @pl.when(it >= 1)
                def _():
                    drain(b2, tsem)

                @pl.when(it + 1 < 40)
                def _():
                    fire_gather(it + 1, b2)
            return carry

        lax.fori_loop(0, 20, body, 0)
        drain(1, tsem)
        plsc.subcore_barrier()
        for z in range(5):
            pltpu.sync_copy(acc.at[pl.ds(s * 640 + z * 128, 128)], r0)
            pltpu.sync_copy(
                r0,
                out_hbm.at[pl.ds((2 * h + c) * NP + s * 640 + z * 128, 128)])
        plsc.subcore_barrier()


@functools.partial(
    pl.kernel,
    out_type=jax.ShapeDtypeStruct((2 * NP, D_OUT), jnp.float32),
    mesh=_mesh,
    compiler_params=_sc_params,
    scratch_types=[
        pltpu.VMEM((80, 128), jnp.int32),
        pltpu.VMEM((80, 128), jnp.int32),
    ] + [pltpu.VMEM((128, D_OUT), jnp.float32)] * 8 + [
        pltpu.VMEM_SHARED((NP, D_OUT), jnp.float32),
        pltpu.SemaphoreType.DMA((8,)),
        pltpu.SemaphoreType.DMA((8,)),
    ],
)
def _agg2_kernel(table_hbm, src4_hbm, dst2_hbm, zeros_hbm, out_hbm,
                 srcb, dstb, r0, r1, r2, r3, r4, r5, r6, r7,
                 acc, gsem, tsem):
    c = lax.axis_index("c")
    s = lax.axis_index("s")
    rows8 = [r0, r1, r2, r3, r4, r5, r6, r7]
    _zero_acc(zeros_hbm, r0, acc, s)
    base = c * 640 + s * 40
    pltpu.sync_copy(src4_hbm.at[0, pl.ds(base, 40)], srcb.at[pl.ds(0, 40)])
    pltpu.sync_copy(dst2_hbm.at[pl.ds(base, 40)], dstb.at[pl.ds(0, 40)])
    plsc.subcore_barrier()
    _agg_pass(table_hbm, zeros_hbm, srcb, dstb, rows8, acc, gsem, tsem,
              nslots=40)
    plsc.subcore_barrier()
    _readout(acc, r0, out_hbm, s, c * NP)


def _mm1_body(x_ref, w_ref, cnt_ref, o_ref):
    s = lax.rsqrt(jnp.maximum(cnt_ref[...], 1.0))
    res = jnp.dot(x_ref[...] * s, w_ref[...],
                  preferred_element_type=jnp.float32)
    for h in range(2):
        o_ref[h] = res[:, h * 128:(h + 1) * 128]


def _mm2_body(a_ref, w_ref, b1_ref, ci_ref, co_ref, o_ref):
    si = lax.rsqrt(jnp.maximum(ci_ref[...], 1.0))
    so = lax.rsqrt(jnp.maximum(co_ref[...], 1.0))
    acc = jnp.zeros(o_ref.shape, jnp.float32)
    for h in range(2):
        t = jnp.maximum((a_ref[h, 0] + a_ref[h, 1]) * si + b1_ref[h],
                        0.0) * so
        acc = acc + jnp.dot(t, w_ref[h], preferred_element_type=jnp.float32)
    o_ref[...] = acc


def _fin_body(p_ref, ci_ref, b2_ref, o_ref):
    si = lax.rsqrt(jnp.maximum(ci_ref[...], 1.0))
    o_ref[...] = (p_ref[0] + p_ref[1]) * si + b2_ref[...]


def kernel(features, edge_index, W1, b1, W2, b2):
    src = edge_index[0]
    dst = edge_index[1]
    pad = EP - E
    fake_dst = N + jnp.arange(pad, dtype=jnp.int32) % (NP - N)
    src_pad = jnp.concatenate([src, jnp.zeros((pad,), jnp.int32)])
    dst_pad = jnp.concatenate([dst, fake_dst])
    src4 = (src_pad[None, :]
            + (jnp.arange(4, dtype=jnp.int32) * N)[:, None]).reshape(
                4, EROWS, 128)
    dst2 = dst_pad.reshape(EROWS, 128)
    edge2 = jnp.concatenate(
        [edge_index, jnp.stack([fake_dst, fake_dst])], axis=1
    ).reshape(2, EROWS, 128)
    zeros_a = jnp.zeros((640, DQ), jnp.float32)
    z128 = jnp.zeros((128, 128), jnp.float32)
    z1d = jnp.zeros((NP,), jnp.float32)

    counts = _deg_kernel(edge2, z1d)
    co_col = counts[0][:, None]
    ci_col = counts[1][:, None]

    h1 = pl.pallas_call(
        _mm1_body,
        grid=(10,),
        in_specs=[
            pl.BlockSpec((1000, D_IN), lambda i: (i, 0)),
            pl.BlockSpec((D_IN, D_H), lambda i: (0, 0)),
            pl.BlockSpec((1000, 1), lambda i: (i, 0)),
        ],
        out_specs=pl.BlockSpec((2, 1000, 128), lambda i: (0, i, 0)),
        out_shape=jax.ShapeDtypeStruct((2, N, 128), jnp.float32),
    )(features, W1, co_col[:N])

    table1 = h1.reshape(2 * N, 128)
    agg1 = _agg1_kernel(table1, src4, dst2, z128)
    agg1 = agg1.reshape(2, 2, NP, 128)

    h2 = pl.pallas_call(
        _mm2_body,
        grid=(10,),
        in_specs=[
            pl.BlockSpec((2, 2, 1024, 128), lambda i: (0, 0, i, 0)),
            pl.BlockSpec((2, 128, D_OUT), lambda i: (0, 0, 0)),
            pl.BlockSpec((2, 1, 128), lambda i: (0, 0, 0)),
            pl.BlockSpec((1024, 1), lambda i: (i, 0)),
            pl.BlockSpec((1024, 1), lambda i: (i, 0)),
        ],
        out_specs=pl.BlockSpec((1024, D_OUT), lambda i: (i, 0)),
        out_shape=jax.ShapeDtypeStruct((NP, D_OUT), jnp.float32),
    )(agg1, W2.reshape(2, 128, D_OUT), b1.reshape(2, 1, 128), ci_col, co_col)

    p = _agg2_kernel(h2, src4, dst2, zeros_a)
    p = p.reshape(2, NP, D_OUT)

    out = pl.pallas_call(
        _fin_body,
        grid=(10,),
        in_specs=[
            pl.BlockSpec((2, 1000, D_OUT), lambda i: (0, i, 0)),
            pl.BlockSpec((1000, 1), lambda i: (i, 0)),
            pl.BlockSpec((1, D_OUT), lambda i: (0, 0)),
        ],
        out_specs=pl.BlockSpec((1000, D_OUT), lambda i: (i, 0)),
        out_shape=jax.ShapeDtypeStruct((N, D_OUT), jnp.float32),
    )(p, ci_col, b2.reshape(1, D_OUT))
    return out


# speedup vs baseline: 1.2318x; 1.2318x over previous
import functools

import jax
import jax.numpy as jnp
from jax import lax
from jax.experimental import pallas as pl
from jax.experimental.pallas import tpu as pltpu
from jax.experimental.pallas import tpu_sc as plsc

N = 10000
E = 160000
D_IN = 256
D_H = 256
D_OUT = 64

NP = 10240
EP = 163840
EROWS = EP // 128
DQ = 64

_mesh = plsc.VectorSubcoreMesh(core_axis_name="c", subcore_axis_name="s")
_sc_params = pltpu.CompilerParams(use_tc_tiling_on_sc=False)


@functools.partial(
    pl.kernel,
    out_type=jax.ShapeDtypeStruct((2, NP), jnp.float32),
    mesh=_mesh,
    compiler_params=_sc_params,
    scratch_types=[
        pltpu.VMEM((4, 128), jnp.int32),
        pltpu.VMEM((128,), jnp.float32),
        pltpu.VMEM((640,), jnp.float32),
        pltpu.VMEM_SHARED((NP,), jnp.float32),
    ],
)
def _deg_kernel(edge_hbm, z1d_hbm, out_hbm, idxv, ones, vbuf, acc):
    c = lax.axis_index("c")
    s = lax.axis_index("s")
    pltpu.sync_copy(z1d_hbm.at[pl.ds(0, 640)], vbuf)
    pltpu.sync_copy(vbuf, acc.at[pl.ds(s * 640, 640)])
    for k in range(8):
        ones[pl.ds(k * 16, 16)] = jnp.ones((16,), jnp.float32)
    plsc.subcore_barrier()

    def body(i, carry):
        pltpu.sync_copy(edge_hbm.at[c, pl.ds(s * 80 + i * 4, 4)], idxv)
        for j in range(4):
            pltpu.sync_copy(ones, acc.at[idxv.at[j]], add=True)
        return carry

    lax.fori_loop(0, 20, body, 0)
    plsc.subcore_barrier()
    pltpu.sync_copy(acc.at[pl.ds(s * 640, 640)], vbuf)
    pltpu.sync_copy(vbuf, out_hbm.at[c, pl.ds(s * 640, 640)])


def _agg_pass(table_hbm, zdrain_hbm, srcb, dstb, rows8, acc, gsem, tsem,
              nslots):
    def fire_gather(slot, b):
        pltpu.async_copy(table_hbm.at[srcb.at[slot]], rows8[b], gsem.at[b])

    def fire_scatter(slot, b):
        pltpu.async_copy(rows8[b], acc.at[dstb.at[slot]], tsem.at[b],
                         add=True)

    def drain(b, sem):
        pltpu.make_async_copy(zdrain_hbm.at[pl.ds(0, 128)], rows8[b],
                              sem.at[b]).wait()

    for b in range(4):
        fire_gather(b, b)

    def body(outer, carry):
        for b in range(8):
            it = outer * 8 + b
            drain(b, gsem)
            fire_scatter(it, b)
            b2 = (b + 4) % 8

            @pl.when(it >= 4)
            def _():
                drain(b2, tsem)

            @pl.when(it + 4 < nslots)
            def _():
                fire_gather(it + 4, b2)
        return carry

    lax.fori_loop(0, nslots // 8, body, 0)
    for b in (4, 5, 6, 7):
        drain(b, tsem)


def _zero_acc(zeros_hbm, zb, acc, s):
    pltpu.sync_copy(zeros_hbm.at[pl.ds(0, 128)], zb)
    for z in range(5):
        pltpu.sync_copy(zb, acc.at[pl.ds(s * 640 + z * 128, 128)])


def _readout(acc, zb, out_hbm, s, out_base):
    for z in range(5):
        pltpu.sync_copy(acc.at[pl.ds(s * 640 + z * 128, 128)], zb)
        pltpu.sync_copy(
            zb, out_hbm.at[pl.ds(out_base + s * 640 + z * 128, 128)])


@functools.partial(
    pl.kernel,
    out_type=jax.ShapeDtypeStruct((2 * NP, 128), jnp.float32),
    mesh=_mesh,
    compiler_params=_sc_params,
    scratch_types=[
        pltpu.VMEM((40, 128), jnp.int32),
        pltpu.VMEM((40, 128), jnp.int32),
        pltpu.VMEM((128, 128), jnp.float32),
        pltpu.VMEM((128, 128), jnp.float32),
        pltpu.VMEM_SHARED((NP, 128), jnp.float32),
        pltpu.SemaphoreType.DMA((2,)),
        pltpu.SemaphoreType.DMA((2,)),
    ],
)
def _agg1_kernel(table_hbm, src4_hbm, dst2_hbm, z128_hbm, out_hbm,
                 srcb, dstb, r0, r1, acc, gsem, tsem):
    c = lax.axis_index("c")
    s = lax.axis_index("s")
    rows2 = [r0, r1]
    pltpu.sync_copy(z128_hbm, r0)
    for z in range(5):
        pltpu.sync_copy(r0, acc.at[pl.ds(s * 640 + z * 128, 128)])
    plsc.subcore_barrier()

    def fire_gather(slot, b):
        pltpu.async_copy(table_hbm.at[srcb.at[slot]], rows2[b], gsem.at[b])

    def drain(b, sem):
        pltpu.make_async_copy(table_hbm.at[pl.ds(0, 128)], rows2[b],
                              sem.at[b]).wait()

    for stage in range(2):
        base = s * 80 + stage * 40
        pltpu.sync_copy(src4_hbm.at[c, pl.ds(base, 40)], srcb)
        pltpu.sync_copy(dst2_hbm.at[pl.ds(base, 40)], dstb)
        fire_gather(0, 0)

        def body(outer, carry):
            for b in range(2):
                it = outer * 2 + b
                drain(b, gsem)
                pltpu.async_copy(rows2[b], acc.at[dstb.at[it]],
                                 tsem.at[b], add=True)
                b2 = 1 - b

                @pl.when(it >= 1)
                def _():
                    drain(b2, tsem)

                @pl.when(it + 1 < 40)
                def _():
                    fire_gather(it + 1, b2)
            return carry

        lax.fori_loop(0, 20, body, 0)
        drain(1, tsem)

    plsc.subcore_barrier()
    for z in range(5):
        pltpu.sync_copy(acc.at[pl.ds(s * 640 + z * 128, 128)], r0)
        pltpu.sync_copy(
            r0, out_hbm.at[pl.ds(c * NP + s * 640 + z * 128, 128)])


@functools.partial(
    pl.kernel,
    out_type=jax.ShapeDtypeStruct((2 * NP, D_OUT), jnp.float32),
    mesh=_mesh,
    compiler_params=_sc_params,
    scratch_types=[
        pltpu.VMEM((80, 128), jnp.int32),
        pltpu.VMEM((80, 128), jnp.int32),
    ] + [pltpu.VMEM((128, D_OUT), jnp.float32)] * 8 + [
        pltpu.VMEM_SHARED((NP, D_OUT), jnp.float32),
        pltpu.SemaphoreType.DMA((8,)),
        pltpu.SemaphoreType.DMA((8,)),
    ],
)
def _agg2_kernel(table_hbm, src4_hbm, dst2_hbm, zeros_hbm, out_hbm,
                 srcb, dstb, r0, r1, r2, r3, r4, r5, r6, r7,
                 acc, gsem, tsem):
    c = lax.axis_index("c")
    s = lax.axis_index("s")
    rows8 = [r0, r1, r2, r3, r4, r5, r6, r7]
    _zero_acc(zeros_hbm, r0, acc, s)
    base = c * 640 + s * 40
    pltpu.sync_copy(src4_hbm.at[2 * c, pl.ds(base, 40)],
                    srcb.at[pl.ds(0, 40)])
    pltpu.sync_copy(dst2_hbm.at[pl.ds(base, 40)], dstb.at[pl.ds(0, 40)])
    plsc.subcore_barrier()
    _agg_pass(table_hbm, zeros_hbm, srcb, dstb, rows8, acc, gsem, tsem,
              nslots=40)
    plsc.subcore_barrier()
    _readout(acc, r0, out_hbm, s, c * NP)


def _mm1_body(x_ref, w_ref, cnt_ref, o_ref):
    s = lax.rsqrt(jnp.maximum(cnt_ref[...], 1.0))
    res = jnp.dot(x_ref[...] * s, w_ref[...],
                  preferred_element_type=jnp.float32)
    for h in range(2):
        o_ref[h] = res[:, h * 128:(h + 1) * 128]


def _mm2_body(a_ref, w_ref, b1_ref, ci_ref, co_ref, o_ref):
    si = lax.rsqrt(jnp.maximum(ci_ref[...], 1.0))
    so = lax.rsqrt(jnp.maximum(co_ref[...], 1.0))
    acc = jnp.zeros(o_ref.shape[1:], jnp.float32)
    for h in range(2):
        t = jnp.maximum(a_ref[h] * si + b1_ref[h], 0.0) * so
        acc = acc + jnp.dot(t, w_ref[h], preferred_element_type=jnp.float32)
    o_ref[0] = acc
    o_ref[1] = acc


def _fin_body(p_ref, ci_ref, b2_ref, o_ref):
    si = lax.rsqrt(jnp.maximum(ci_ref[...], 1.0))
    o_ref[...] = (p_ref[0] + p_ref[1]) * si + b2_ref[...]


def kernel(features, edge_index, W1, b1, W2, b2):
    src = edge_index[0]
    dst = edge_index[1]
    pad = EP - E
    fake_dst = N + jnp.arange(pad, dtype=jnp.int32) % (NP - N)
    src_pad = jnp.concatenate([src, jnp.zeros((pad,), jnp.int32)])
    dst_pad = jnp.concatenate([dst, fake_dst])
    shifts = jnp.array([0, N, NP], dtype=jnp.int32)
    src4 = (src_pad[None, :] + shifts[:, None]).reshape(3, EROWS, 128)
    dst2 = dst_pad.reshape(EROWS, 128)
    edge2 = jnp.concatenate(
        [edge_index, jnp.stack([fake_dst, fake_dst])], axis=1
    ).reshape(2, EROWS, 128)
    zeros_a = jnp.zeros((640, DQ), jnp.float32)
    z128 = jnp.zeros((128, 128), jnp.float32)
    z1d = jnp.zeros((NP,), jnp.float32)

    counts = _deg_kernel(edge2, z1d)
    co_col = counts[0][:, None]
    ci_col = counts[1][:, None]

    h1 = pl.pallas_call(
        _mm1_body,
        grid=(10,),
        in_specs=[
            pl.BlockSpec((1000, D_IN), lambda i: (i, 0)),
            pl.BlockSpec((D_IN, D_H), lambda i: (0, 0)),
            pl.BlockSpec((1000, 1), lambda i: (i, 0)),
        ],
        out_specs=pl.BlockSpec((2, 1000, 128), lambda i: (0, i, 0)),
        out_shape=jax.ShapeDtypeStruct((2, N, 128), jnp.float32),
    )(features, W1, co_col[:N])

    table1 = h1.reshape(2 * N, 128)
    agg1 = _agg1_kernel(table1, src4, dst2, z128)
    agg1 = agg1.reshape(2, NP, 128)

    h2 = pl.pallas_call(
        _mm2_body,
        grid=(10,),
        in_specs=[
            pl.BlockSpec((2, 1024, 128), lambda i: (0, i, 0)),
            pl.BlockSpec((2, 128, D_OUT), lambda i: (0, 0, 0)),
            pl.BlockSpec((2, 1, 128), lambda i: (0, 0, 0)),
            pl.BlockSpec((1024, 1), lambda i: (i, 0)),
            pl.BlockSpec((1024, 1), lambda i: (i, 0)),
        ],
        out_specs=pl.BlockSpec((2, 1024, D_OUT), lambda i: (0, i, 0)),
        out_shape=jax.ShapeDtypeStruct((2, NP, D_OUT), jnp.float32),
    )(agg1, W2.reshape(2, 128, D_OUT), b1.reshape(2, 1, 128), ci_col, co_col)

    p = _agg2_kernel(h2.reshape(2 * NP, D_OUT), src4, dst2, zeros_a)
    p = p.reshape(2, NP, D_OUT)

    out = pl.pallas_call(
        _fin_body,
        grid=(10,),
        in_specs=[
            pl.BlockSpec((2, 1000, D_OUT), lambda i: (0, i, 0)),
            pl.BlockSpec((1000, 1), lambda i: (i, 0)),
            pl.BlockSpec((1, D_OUT), lambda i: (0, 0)),
        ],
        out_specs=pl.BlockSpec((1000, D_OUT), lambda i: (i, 0)),
        out_shape=jax.ShapeDtypeStruct((N, D_OUT), jnp.float32),
    )(p, ci_col, b2.reshape(1, D_OUT))
    return out


# speedup vs baseline: 1.3382x; 1.0864x over previous
import functools

import jax
import jax.numpy as jnp
from jax import lax
from jax.experimental import pallas as pl
from jax.experimental.pallas import tpu as pltpu
from jax.experimental.pallas import tpu_sc as plsc

N = 10000
E = 160000
D_IN = 256
D_H = 256
D_OUT = 64

NP = 10240
EP = 163840
EROWS = EP // 128
DQ = 64

_mesh = plsc.VectorSubcoreMesh(core_axis_name="c", subcore_axis_name="s")
_sc_params = pltpu.CompilerParams(use_tc_tiling_on_sc=False)


@functools.partial(
    pl.kernel,
    out_type=jax.ShapeDtypeStruct((2, NP), jnp.float32),
    mesh=_mesh,
    compiler_params=_sc_params,
    scratch_types=[
        pltpu.VMEM((4, 128), jnp.int32),
        pltpu.VMEM((128,), jnp.float32),
        pltpu.VMEM((640,), jnp.float32),
        pltpu.VMEM_SHARED((NP,), jnp.float32),
    ],
)
def _deg_kernel(edge_hbm, z1d_hbm, out_hbm, idxv, ones, vbuf, acc):
    c = lax.axis_index("c")
    s = lax.axis_index("s")
    pltpu.sync_copy(z1d_hbm.at[pl.ds(0, 640)], vbuf)
    pltpu.sync_copy(vbuf, acc.at[pl.ds(s * 640, 640)])
    for k in range(8):
        ones[pl.ds(k * 16, 16)] = jnp.ones((16,), jnp.float32)
    plsc.subcore_barrier()

    def body(i, carry):
        pltpu.sync_copy(edge_hbm.at[c, pl.ds(s * 80 + i * 4, 4)], idxv)
        for j in range(4):
            pltpu.sync_copy(ones, acc.at[idxv.at[j]], add=True)
        return carry

    lax.fori_loop(0, 20, body, 0)
    plsc.subcore_barrier()
    pltpu.sync_copy(acc.at[pl.ds(s * 640, 640)], vbuf)
    pltpu.sync_copy(vbuf, out_hbm.at[c, pl.ds(s * 640, 640)])


def _agg_pass(table_hbm, zdrain_hbm, srcb, dstb, rows8, acc, gsem, tsem,
              nslots):
    def fire_gather(slot, b):
        pltpu.async_copy(table_hbm.at[srcb.at[slot]], rows8[b], gsem.at[b])

    def fire_scatter(slot, b):
        pltpu.async_copy(rows8[b], acc.at[dstb.at[slot]], tsem.at[b],
                         add=True)

    def drain(b, sem):
        pltpu.make_async_copy(zdrain_hbm.at[pl.ds(0, 128)], rows8[b],
                              sem.at[b]).wait()

    for b in range(4):
        fire_gather(b, b)

    def body(outer, carry):
        for b in range(8):
            it = outer * 8 + b
            drain(b, gsem)
            fire_scatter(it, b)
            b2 = (b + 4) % 8

            @pl.when(it >= 4)
            def _():
                drain(b2, tsem)

            @pl.when(it + 4 < nslots)
            def _():
                fire_gather(it + 4, b2)
        return carry

    lax.fori_loop(0, nslots // 8, body, 0)
    for b in (4, 5, 6, 7):
        drain(b, tsem)


def _zero_acc(zeros_hbm, zb, acc, s):
    pltpu.sync_copy(zeros_hbm.at[pl.ds(0, 128)], zb)
    for z in range(5):
        pltpu.sync_copy(zb, acc.at[pl.ds(s * 640 + z * 128, 128)])


def _readout(acc, zb, out_hbm, s, out_base):
    for z in range(5):
        pltpu.sync_copy(acc.at[pl.ds(s * 640 + z * 128, 128)], zb)
        pltpu.sync_copy(
            zb, out_hbm.at[pl.ds(out_base + s * 640 + z * 128, 128)])


@functools.partial(
    pl.kernel,
    out_type=jax.ShapeDtypeStruct((2 * NP, 128), jnp.float32),
    mesh=_mesh,
    compiler_params=_sc_params,
    scratch_types=[
        pltpu.VMEM((40, 128), jnp.int32),
        pltpu.VMEM((40, 128), jnp.int32),
        pltpu.VMEM((128, 128), jnp.float32),
        pltpu.VMEM((128, 128), jnp.float32),
        pltpu.VMEM_SHARED((NP, 128), jnp.float32),
        pltpu.SemaphoreType.DMA((2,)),
        pltpu.SemaphoreType.DMA((2,)),
    ],
)
def _agg1_kernel(table_hbm, src4_hbm, dst2_hbm, z128_hbm, out_hbm,
                 srcb, dstb, r0, r1, acc, gsem, tsem):
    c = lax.axis_index("c")
    s = lax.axis_index("s")
    rows2 = [r0, r1]
    pltpu.sync_copy(z128_hbm, r0)
    for z in range(5):
        pltpu.sync_copy(r0, acc.at[pl.ds(s * 640 + z * 128, 128)])
    plsc.subcore_barrier()

    def fire_gather(slot, b):
        pltpu.async_copy(table_hbm.at[srcb.at[slot]], rows2[b], gsem.at[b])

    def drain(b, sem):
        pltpu.make_async_copy(table_hbm.at[pl.ds(0, 128)], rows2[b],
                              sem.at[b]).wait()

    for stage in range(2):
        base = s * 80 + stage * 40
        pltpu.sync_copy(src4_hbm.at[c, pl.ds(base, 40)], srcb)
        pltpu.sync_copy(dst2_hbm.at[pl.ds(base, 40)], dstb)
        fire_gather(0, 0)

        def body(outer, carry):
            for b in range(2):
                it = outer * 2 + b
                drain(b, gsem)
                pltpu.async_copy(rows2[b], acc.at[dstb.at[it]],
                                 tsem.at[b], add=True)
                b2 = 1 - b

                @pl.when(it >= 1)
                def _():
                    drain(b2, tsem)

                @pl.when(it + 1 < 40)
                def _():
                    fire_gather(it + 1, b2)
            return carry

        lax.fori_loop(0, 20, body, 0)
        drain(1, tsem)

    plsc.subcore_barrier()
    for z in range(5):
        pltpu.sync_copy(acc.at[pl.ds(s * 640 + z * 128, 128)], r0)
        pltpu.sync_copy(
            r0, out_hbm.at[pl.ds(c * NP + s * 640 + z * 128, 128)])


@functools.partial(
    pl.kernel,
    out_type=jax.ShapeDtypeStruct((2 * NP, D_OUT), jnp.float32),
    mesh=_mesh,
    compiler_params=_sc_params,
    scratch_types=[
        pltpu.VMEM((80, 128), jnp.int32),
        pltpu.VMEM((80, 128), jnp.int32),
    ] + [pltpu.VMEM((128, D_OUT), jnp.float32)] * 8 + [
        pltpu.VMEM_SHARED((NP, D_OUT), jnp.float32),
        pltpu.SemaphoreType.DMA((8,)),
        pltpu.SemaphoreType.DMA((8,)),
    ],
)
def _agg2_kernel(table_hbm, src4_hbm, dst2_hbm, zeros_hbm, out_hbm,
                 srcb, dstb, r0, r1, r2, r3, r4, r5, r6, r7,
                 acc, gsem, tsem):
    c = lax.axis_index("c")
    s = lax.axis_index("s")
    rows8 = [r0, r1, r2, r3, r4, r5, r6, r7]
    _zero_acc(zeros_hbm, r0, acc, s)
    base = c * 640 + s * 40
    pltpu.sync_copy(src4_hbm.at[0, pl.ds(base, 40)], srcb.at[pl.ds(0, 40)])
    pltpu.sync_copy(dst2_hbm.at[pl.ds(base, 40)], dstb.at[pl.ds(0, 40)])
    plsc.subcore_barrier()
    _agg_pass(table_hbm, zeros_hbm, srcb, dstb, rows8, acc, gsem, tsem,
              nslots=40)
    plsc.subcore_barrier()
    _readout(acc, r0, out_hbm, s, c * NP)


def _mm1_body(x_ref, w_ref, cnt_ref, o_ref):
    s = lax.rsqrt(jnp.maximum(cnt_ref[...], 1.0))
    res = jnp.dot(x_ref[...] * s, w_ref[...],
                  preferred_element_type=jnp.float32)
    for h in range(2):
        o_ref[h] = res[:, h * 128:(h + 1) * 128]


def _mm2_body(a_ref, w_ref, b1_ref, ci_ref, co_ref, o_ref):
    si = lax.rsqrt(jnp.maximum(ci_ref[...], 1.0))
    so = lax.rsqrt(jnp.maximum(co_ref[...], 1.0))
    acc = jnp.zeros(o_ref.shape, jnp.float32)
    for h in range(2):
        t = jnp.maximum(a_ref[h] * si + b1_ref[h], 0.0) * so
        acc = acc + jnp.dot(t, w_ref[h], preferred_element_type=jnp.float32)
    o_ref[...] = acc


def _fin_body(p_ref, ci_ref, b2_ref, o_ref):
    si = lax.rsqrt(jnp.maximum(ci_ref[...], 1.0))
    o_ref[...] = (p_ref[0] + p_ref[1]) * si + b2_ref[...]


def kernel(features, edge_index, W1, b1, W2, b2):
    src = edge_index[0]
    dst = edge_index[1]
    pad = EP - E
    fake_dst = N + jnp.arange(pad, dtype=jnp.int32) % (NP - N)
    src_pad = jnp.concatenate([src, jnp.zeros((pad,), jnp.int32)])
    dst_pad = jnp.concatenate([dst, fake_dst])
    src4 = (src_pad[None, :]
            + (jnp.arange(4, dtype=jnp.int32) * N)[:, None]).reshape(
                4, EROWS, 128)
    dst2 = dst_pad.reshape(EROWS, 128)
    edge2 = jnp.concatenate(
        [edge_index, jnp.stack([fake_dst, fake_dst])], axis=1
    ).reshape(2, EROWS, 128)
    zeros_a = jnp.zeros((640, DQ), jnp.float32)
    z128 = jnp.zeros((128, 128), jnp.float32)
    z1d = jnp.zeros((NP,), jnp.float32)

    counts = _deg_kernel(edge2, z1d)
    co_col = counts[0][:, None]
    ci_col = counts[1][:, None]

    h1 = pl.pallas_call(
        _mm1_body,
        grid=(10,),
        in_specs=[
            pl.BlockSpec((1000, D_IN), lambda i: (i, 0)),
            pl.BlockSpec((D_IN, D_H), lambda i: (0, 0)),
            pl.BlockSpec((1000, 1), lambda i: (i, 0)),
        ],
        out_specs=pl.BlockSpec((2, 1000, 128), lambda i: (0, i, 0)),
        out_shape=jax.ShapeDtypeStruct((2, N, 128), jnp.float32),
    )(features, W1, co_col[:N])

    table1 = h1.reshape(2 * N, 128)
    agg1 = _agg1_kernel(table1, src4, dst2, z128)
    agg1 = agg1.reshape(2, NP, 128)

    h2 = pl.pallas_call(
        _mm2_body,
        grid=(10,),
        in_specs=[
            pl.BlockSpec((2, 1024, 128), lambda i: (0, i, 0)),
            pl.BlockSpec((2, 128, D_OUT), lambda i: (0, 0, 0)),
            pl.BlockSpec((2, 1, 128), lambda i: (0, 0, 0)),
            pl.BlockSpec((1024, 1), lambda i: (i, 0)),
            pl.BlockSpec((1024, 1), lambda i: (i, 0)),
        ],
        out_specs=pl.BlockSpec((1024, D_OUT), lambda i: (i, 0)),
        out_shape=jax.ShapeDtypeStruct((NP, D_OUT), jnp.float32),
    )(agg1, W2.reshape(2, 128, D_OUT), b1.reshape(2, 1, 128), ci_col, co_col)

    p = _agg2_kernel(h2, src4, dst2, zeros_a)
    p = p.reshape(2, NP, D_OUT)

    out = pl.pallas_call(
        _fin_body,
        grid=(10,),
        in_specs=[
            pl.BlockSpec((2, 1000, D_OUT), lambda i: (0, i, 0)),
            pl.BlockSpec((1000, 1), lambda i: (i, 0)),
            pl.BlockSpec((1, D_OUT), lambda i: (0, 0)),
        ],
        out_specs=pl.BlockSpec((1000, D_OUT), lambda i: (i, 0)),
        out_shape=jax.ShapeDtypeStruct((N, D_OUT), jnp.float32),
    )(p, ci_col, b2.reshape(1, D_OUT))
    return out
